# baseline (speedup 1.0000x reference)
import functools

import jax
import jax.numpy as jnp
from jax import lax
from jax.experimental import pallas as pl
from jax.experimental.pallas import tpu as pltpu
from jax.experimental.pallas import tpu_sc as plsc

N_FIELDS = 26
VOCAB = 1000000
BATCH = 16384
NC = 2
NS = 16
L = 16
NW = NC * NS
RPW = BATCH // NW
IPW = N_FIELDS * RPW
CPR = RPW // L

_mesh = plsc.VectorSubcoreMesh(core_axis_name="c", subcore_axis_name="s")


@functools.partial(
    pl.kernel,
    mesh=_mesh,
    out_type=jax.ShapeDtypeStruct((BATCH,), jnp.float32),
    scratch_types=[
        pltpu.VMEM((IPW,), jnp.int32),
        pltpu.VMEM((IPW,), jnp.float32),
        pltpu.VMEM((RPW,), jnp.float32),
        pltpu.SemaphoreType.DMA,
    ],
)
def _sc_lookup(xtw_hbm, table_hbm, out_hbm, idx_v, vals_v, out_v, sem):
    wid = lax.axis_index("s") * NC + lax.axis_index("c")

    pltpu.sync_copy(xtw_hbm.at[wid], idx_v)

    def idx_body(i, carry):
        s = pl.multiple_of(i * L, L)
        idx_v[pl.ds(s, L)] = idx_v[pl.ds(s, L)] + (i // CPR) * VOCAB
        return carry

    lax.fori_loop(0, IPW // L, idx_body, 0)

    pltpu.async_copy(table_hbm.at[idx_v], vals_v, sem).wait()

    def red_body(c, carry):
        s = pl.multiple_of(c * L, L)
        acc = vals_v[pl.ds(s, L)]
        for f in range(1, N_FIELDS):
            acc = acc + vals_v[pl.ds(f * RPW + s, L)]
        out_v[pl.ds(s, L)] = 1.0 / (1.0 + jnp.exp(-acc))
        return carry

    lax.fori_loop(0, CPR, red_body, 0)

    pltpu.sync_copy(out_v, out_hbm.at[pl.ds(wid * RPW, RPW)])


def kernel(X, lin_table):
    xtw = X.T.reshape(N_FIELDS, NW, RPW).transpose(1, 0, 2).reshape(NW, IPW)
    out = _sc_lookup(xtw, lin_table.reshape(-1))
    return out.reshape(BATCH, 1)


# speedup vs baseline: 20.6847x; 20.6847x over previous
import functools

import jax
import jax.numpy as jnp
from jax import lax
from jax.experimental import pallas as pl
from jax.experimental.pallas import tpu as pltpu
from jax.experimental.pallas import tpu_sc as plsc

N_FIELDS = 26
VOCAB = 1000000
BATCH = 16384
NC = 2
NS = 16
L = 16
FPC = N_FIELDS // NC
RPT = BATCH // NS

ALIGNED = 999936
TAIL_IN = VOCAB - 128
ST_STRIDE = 62464
ST_WIDTH = 62976
TAILBASE = 2 * ALIGNED
S_WORDS = TAILBASE + N_FIELDS * 128
TAIL_CHUNK = (N_FIELDS * 128) // NS

_mesh = plsc.VectorSubcoreMesh(core_axis_name="c", subcore_axis_name="s")


@functools.partial(
    pl.kernel,
    mesh=_mesh,
    out_type=jax.ShapeDtypeStruct((NC * BATCH,), jnp.float32),
    scratch_types=[
        pltpu.VMEM_SHARED((S_WORDS,), jnp.float32),
        pltpu.VMEM((RPT,), jnp.int32),
        pltpu.VMEM((RPT,), jnp.float32),
        pltpu.VMEM((RPT,), jnp.float32),
        pltpu.SemaphoreType.DMA,
        pltpu.SemaphoreType.DMA,
    ],
)
def _sc_partial(xt_hbm, table_hbm, tail_hbm, out_hbm, smem, idx_v, vals_v,
                acc_v, stage_sem, gather_sem):
    cid = lax.axis_index("c")
    sid = lax.axis_index("s")
    col = sid * ST_STRIDE

    def fire_stage(j):
        base = (j % 2) * ALIGNED
        pltpu.async_copy(
            table_hbm.at[cid * FPC + j].at[pl.ds(col, ST_WIDTH)],
            smem.at[pl.ds(base + col, ST_WIDTH)], stage_sem)

    def wait_stage(j):
        base = (j % 2) * ALIGNED
        pltpu.make_async_copy(
            table_hbm.at[0].at[pl.ds(col, ST_WIDTH)],
            smem.at[pl.ds(base + col, ST_WIDTH)], stage_sem).wait()

    pltpu.sync_copy(tail_hbm.at[pl.ds(sid * TAIL_CHUNK, TAIL_CHUNK)],
                    vals_v.at[pl.ds(0, TAIL_CHUNK)])
    pltpu.sync_copy(vals_v.at[pl.ds(0, TAIL_CHUNK)],
                    smem.at[pl.ds(TAILBASE + sid * TAIL_CHUNK, TAIL_CHUNK)])
    fire_stage(0)
    wait_stage(0)
    plsc.subcore_barrier()

    def init_body(c, carry):
        s = pl.multiple_of(c * L, L)
        acc_v[pl.ds(s, L)] = jnp.zeros((L,), jnp.float32)
        return carry

    lax.fori_loop(0, RPT // L, init_body, 0)

    for j in range(FPC):
        if j + 1 < FPC:
            fire_stage(j + 1)
        f = cid * FPC + j
        pltpu.sync_copy(
            xt_hbm.at[pl.ds((f * BATCH) + sid * RPT, RPT)], idx_v)
        base = (j % 2) * ALIGNED
        tail_shift = TAILBASE - TAIL_IN + f * 128

        def remap_body(c, carry):
            s = pl.multiple_of(c * L, L)
            v = idx_v[pl.ds(s, L)]
            shift = jnp.where(v >= ALIGNED, tail_shift, base)
            idx_v[pl.ds(s, L)] = v + shift
            return carry

        lax.fori_loop(0, RPT // L, remap_body, 0)
        pltpu.async_copy(smem.at[idx_v], vals_v, gather_sem).wait()

        def red_body(c, carry):
            s = pl.multiple_of(c * L, L)
            acc_v[pl.ds(s, L)] = acc_v[pl.ds(s, L)] + vals_v[pl.ds(s, L)]
            return carry

        lax.fori_loop(0, RPT // L, red_body, 0)
        if j + 1 < FPC:
            wait_stage(j + 1)
            plsc.subcore_barrier()

    pltpu.sync_copy(acc_v, out_hbm.at[pl.ds(cid * BATCH + sid * RPT, RPT)])


def _combine_body(partial_ref, out_ref):
    s = jnp.sum(partial_ref[...], axis=0, keepdims=True)
    out_ref[...] = 1.0 / (1.0 + jnp.exp(-s))


_combine = pl.pallas_call(
    _combine_body,
    out_shape=jax.ShapeDtypeStruct((1, BATCH), jnp.float32),
)


def kernel(X, lin_table):
    xt = X.T.reshape(-1)
    tail = lax.slice(lin_table, (0, TAIL_IN), (N_FIELDS, VOCAB)).reshape(-1)
    partial = _sc_partial(xt, lin_table, tail).reshape(NC, BATCH)
    return _combine(partial).reshape(BATCH, 1)


# speedup vs baseline: 20.9189x; 1.0113x over previous
import functools

import jax
import jax.numpy as jnp
from jax import lax
from jax.experimental import pallas as pl
from jax.experimental.pallas import tpu as pltpu
from jax.experimental.pallas import tpu_sc as plsc

N_FIELDS = 26
VOCAB = 1000000
BATCH = 16384
NC = 2
NS = 16
L = 16
FPC = N_FIELDS // NC
RPT = BATCH // NS

ALIGNED = 999936
TAIL_IN = VOCAB - 128
ST_STRIDE = 62464
ST_WIDTH = 62976
TAILBASE = 2 * ALIGNED
S_WORDS = TAILBASE + N_FIELDS * 128
TAIL_CHUNK = (N_FIELDS * 128) // NS

_mesh = plsc.VectorSubcoreMesh(core_axis_name="c", subcore_axis_name="s")


@functools.partial(
    pl.kernel,
    mesh=_mesh,
    out_type=jax.ShapeDtypeStruct((NC * BATCH,), jnp.float32),
    scratch_types=[
        pltpu.VMEM_SHARED((S_WORDS,), jnp.float32),
        pltpu.VMEM((RPT,), jnp.int32),
        pltpu.VMEM((RPT,), jnp.int32),
        pltpu.VMEM((RPT,), jnp.float32),
        pltpu.VMEM((RPT,), jnp.float32),
        pltpu.SemaphoreType.DMA,
        pltpu.SemaphoreType.DMA,
        pltpu.SemaphoreType.DMA,
    ],
)
def _sc_partial(xt_hbm, table_hbm, tail_hbm, out_hbm, smem, idxA, idxB,
                vals_v, acc_v, stage_sem, gather_sem, idx_sem):
    cid = lax.axis_index("c")
    sid = lax.axis_index("s")
    col = sid * ST_STRIDE
    idxs = (idxA, idxB)

    def fire_idx(j):
        f = cid * FPC + j
        pltpu.async_copy(
            xt_hbm.at[pl.ds(f * BATCH + sid * RPT, RPT)], idxs[j % 2], idx_sem)

    def wait_idx(j):
        pltpu.make_async_copy(
            xt_hbm.at[pl.ds(sid * RPT, RPT)], idxs[j % 2], idx_sem).wait()

    def fire_stage(j):
        base = (j % 2) * ALIGNED
        pltpu.async_copy(
            table_hbm.at[cid * FPC + j].at[pl.ds(col, ST_WIDTH)],
            smem.at[pl.ds(base + col, ST_WIDTH)], stage_sem)

    def wait_stage(j):
        base = (j % 2) * ALIGNED
        pltpu.make_async_copy(
            table_hbm.at[0].at[pl.ds(col, ST_WIDTH)],
            smem.at[pl.ds(base + col, ST_WIDTH)], stage_sem).wait()

    fire_stage(0)
    fire_idx(0)
    pltpu.sync_copy(tail_hbm.at[pl.ds(sid * TAIL_CHUNK, TAIL_CHUNK)],
                    vals_v.at[pl.ds(0, TAIL_CHUNK)])
    pltpu.sync_copy(vals_v.at[pl.ds(0, TAIL_CHUNK)],
                    smem.at[pl.ds(TAILBASE + sid * TAIL_CHUNK, TAIL_CHUNK)])
    wait_stage(0)
    plsc.subcore_barrier()

    def init_body(c, carry):
        s = pl.multiple_of(c * L, L)
        acc_v[pl.ds(s, L)] = jnp.zeros((L,), jnp.float32)
        return carry

    lax.fori_loop(0, RPT // L, init_body, 0)

    for j in range(FPC):
        if j + 1 < FPC:
            fire_stage(j + 1)
            fire_idx(j + 1)
        f = cid * FPC + j
        idx_v = idxs[j % 2]
        wait_idx(j)
        base = (j % 2) * ALIGNED
        tail_shift = TAILBASE - TAIL_IN + f * 128

        def remap_body(c, carry):
            s = pl.multiple_of(c * L, L)
            v = idx_v[pl.ds(s, L)]
            shift = jnp.where(v >= ALIGNED, tail_shift, base)
            idx_v[pl.ds(s, L)] = v + shift
            return carry

        lax.fori_loop(0, RPT // L, remap_body, 0)
        pltpu.async_copy(smem.at[idx_v], vals_v, gather_sem).wait()

        def red_body(c, carry):
            s = pl.multiple_of(c * L, L)
            acc_v[pl.ds(s, L)] = acc_v[pl.ds(s, L)] + vals_v[pl.ds(s, L)]
            return carry

        lax.fori_loop(0, RPT // L, red_body, 0)
        if j + 1 < FPC:
            wait_stage(j + 1)
            plsc.subcore_barrier()

    pltpu.sync_copy(acc_v, out_hbm.at[pl.ds(cid * BATCH + sid * RPT, RPT)])


def _combine_body(partial_ref, out_ref):
    s = jnp.sum(partial_ref[...], axis=0, keepdims=True)
    out_ref[...] = 1.0 / (1.0 + jnp.exp(-s))


_combine = pl.pallas_call(
    _combine_body,
    out_shape=jax.ShapeDtypeStruct((1, BATCH), jnp.float32),
)


def kernel(X, lin_table):
    xt = X.T.reshape(-1)
    tail = lax.slice(lin_table, (0, TAIL_IN), (N_FIELDS, VOCAB)).reshape(-1)
    partial = _sc_partial(xt, lin_table, tail).reshape(NC, BATCH)
    return _combine(partial).reshape(BATCH, 1)
